# baseline (speedup 1.0000x reference)
import jax
import jax.numpy as jnp
from jax.experimental import pallas as pl
from jax.experimental.pallas import tpu as pltpu

N = 10000
D = 128
TM = 200
TM2 = 2000
EPS = 1e-5


def _stage1(h_full_ref, adj_ref, h_row_ref, w1_ref, b1_ref, z1_ref, stats_ref):
    i = pl.program_id(0)
    pooled = jnp.dot(adj_ref[...].astype(jnp.bfloat16),
                     h_full_ref[...].astype(jnp.bfloat16),
                     preferred_element_type=jnp.float32)
    pooled = pooled + h_row_ref[...]
    z1 = jnp.dot(pooled, w1_ref[...],
                 preferred_element_type=jnp.float32) + b1_ref[...]
    z1_ref[...] = z1

    @pl.when(i == 0)
    def _init():
        stats_ref[...] = jnp.zeros_like(stats_ref)

    stats_ref[0:1, :] += jnp.sum(z1, axis=0, keepdims=True)
    stats_ref[1:2, :] += jnp.sum(z1 * z1, axis=0, keepdims=True)


def _stage23(z1_ref, stats1_ref, g1_ref, be1_ref, w2_ref, b2_ref,
             g2_ref, be2_ref, out_ref, z2_scratch, stats2_scratch):
    p = pl.program_id(0)
    i = pl.program_id(1)

    @pl.when(p == 0)
    def _phase0():
        mean = stats1_ref[0:1, :] * (1.0 / N)
        var = stats1_ref[1:2, :] * (1.0 / N) - mean * mean
        scale = g1_ref[...] * jax.lax.rsqrt(var + EPS)
        shift = be1_ref[...] - mean * scale
        a = jnp.maximum(z1_ref[...] * scale + shift, 0.0)
        z2 = jnp.dot(a, w2_ref[...],
                     preferred_element_type=jnp.float32) + b2_ref[...]
        z2_scratch[pl.ds(i * TM2, TM2), :] = z2

        @pl.when(i == 0)
        def _init():
            stats2_scratch[...] = jnp.zeros_like(stats2_scratch)

        stats2_scratch[0:1, :] += jnp.sum(z2, axis=0, keepdims=True)
        stats2_scratch[1:2, :] += jnp.sum(z2 * z2, axis=0, keepdims=True)

    @pl.when(p == 1)
    def _phase1():
        mean = stats2_scratch[0:1, :] * (1.0 / N)
        var = stats2_scratch[1:2, :] * (1.0 / N) - mean * mean
        scale = g2_ref[...] * jax.lax.rsqrt(var + EPS)
        shift = be2_ref[...] - mean * scale
        z2 = z2_scratch[pl.ds(i * TM2, TM2), :]
        out_ref[...] = jnp.maximum(z2 * scale + shift, 0.0)


def kernel(h, Adj_block, padded_neighbor_list, W1, b1, bn1_gamma, bn1_beta,
           W2, b2, bn2_gamma, bn2_beta):
    del padded_neighbor_list
    b1r = b1.reshape(1, D)
    b2r = b2.reshape(1, D)
    g1 = bn1_gamma.reshape(1, D)
    be1 = bn1_beta.reshape(1, D)
    g2 = bn2_gamma.reshape(1, D)
    be2 = bn2_beta.reshape(1, D)

    z1, stats1 = pl.pallas_call(
        _stage1,
        grid=(N // TM,),
        in_specs=[
            pl.BlockSpec((N, D), lambda i: (0, 0)),
            pl.BlockSpec((TM, N), lambda i: (i, 0)),
            pl.BlockSpec((TM, D), lambda i: (i, 0)),
            pl.BlockSpec((D, D), lambda i: (0, 0)),
            pl.BlockSpec((1, D), lambda i: (0, 0)),
        ],
        out_specs=[
            pl.BlockSpec((TM, D), lambda i: (i, 0)),
            pl.BlockSpec((2, D), lambda i: (0, 0)),
        ],
        out_shape=[
            jax.ShapeDtypeStruct((N, D), jnp.float32),
            jax.ShapeDtypeStruct((2, D), jnp.float32),
        ],
    )(h, Adj_block, h, W1, b1r)

    nsteps = N // TM2
    out = pl.pallas_call(
        _stage23,
        grid=(2, nsteps),
        in_specs=[
            pl.BlockSpec((TM2, D), lambda p, i: (jnp.where(p == 0, i, nsteps - 1), 0)),
            pl.BlockSpec((2, D), lambda p, i: (0, 0)),
            pl.BlockSpec((1, D), lambda p, i: (0, 0)),
            pl.BlockSpec((1, D), lambda p, i: (0, 0)),
            pl.BlockSpec((D, D), lambda p, i: (0, 0)),
            pl.BlockSpec((1, D), lambda p, i: (0, 0)),
            pl.BlockSpec((1, D), lambda p, i: (0, 0)),
            pl.BlockSpec((1, D), lambda p, i: (0, 0)),
        ],
        out_specs=pl.BlockSpec((TM2, D), lambda p, i: (jnp.where(p == 0, 0, i), 0)),
        out_shape=jax.ShapeDtypeStruct((N, D), jnp.float32),
        scratch_shapes=[
            pltpu.VMEM((N, D), jnp.float32),
            pltpu.VMEM((2, D), jnp.float32),
        ],
    )(z1, stats1, g1, be1, W2, b2r, g2, be2)

    return out


# speedup vs baseline: 1.0886x; 1.0886x over previous
import jax
import jax.numpy as jnp
from jax.experimental import pallas as pl
from jax.experimental.pallas import tpu as pltpu

N = 10000
D = 128
TM = 400
TM2 = 2000
S1 = N // TM
S2 = N // TM2
EPS = 1e-5


def _gin_kernel(h_full_ref, adj_ref, w1_ref, b1_ref, g1_ref, be1_ref,
                w2_ref, b2_ref, g2_ref, be2_ref, out_ref,
                z1_scratch, z2_scratch, stats_scratch):
    t = pl.program_id(0)

    @pl.when(t < S1)
    def _phase_matmul():
        pooled = jnp.dot(adj_ref[...].astype(jnp.bfloat16),
                         h_full_ref[...].astype(jnp.bfloat16),
                         preferred_element_type=jnp.float32)
        pooled = pooled + h_full_ref[pl.ds(t * TM, TM), :]
        z1 = jnp.dot(pooled, w1_ref[...],
                     preferred_element_type=jnp.float32) + b1_ref[...]
        z1_scratch[pl.ds(t * TM, TM), :] = z1

        @pl.when(t == 0)
        def _init():
            stats_scratch[...] = jnp.zeros_like(stats_scratch)

        stats_scratch[0:1, :] += jnp.sum(z1, axis=0, keepdims=True)
        stats_scratch[1:2, :] += jnp.sum(z1 * z1, axis=0, keepdims=True)

    @pl.when(jnp.logical_and(t >= S1, t < S1 + S2))
    def _phase_mlp():
        i = t - S1
        mean = stats_scratch[0:1, :] * (1.0 / N)
        var = stats_scratch[1:2, :] * (1.0 / N) - mean * mean
        scale = g1_ref[...] * jax.lax.rsqrt(var + EPS)
        shift = be1_ref[...] - mean * scale
        a = jnp.maximum(z1_scratch[pl.ds(i * TM2, TM2), :] * scale + shift,
                        0.0)
        z2 = jnp.dot(a, w2_ref[...],
                     preferred_element_type=jnp.float32) + b2_ref[...]
        z2_scratch[pl.ds(i * TM2, TM2), :] = z2

        stats_scratch[2:3, :] += jnp.sum(z2, axis=0, keepdims=True)
        stats_scratch[3:4, :] += jnp.sum(z2 * z2, axis=0, keepdims=True)

    @pl.when(t >= S1 + S2)
    def _phase_norm():
        i = t - (S1 + S2)
        mean = stats_scratch[2:3, :] * (1.0 / N)
        var = stats_scratch[3:4, :] * (1.0 / N) - mean * mean
        scale = g2_ref[...] * jax.lax.rsqrt(var + EPS)
        shift = be2_ref[...] - mean * scale
        z2 = z2_scratch[pl.ds(i * TM2, TM2), :]
        out_ref[...] = jnp.maximum(z2 * scale + shift, 0.0)


def kernel(h, Adj_block, padded_neighbor_list, W1, b1, bn1_gamma, bn1_beta,
           W2, b2, bn2_gamma, bn2_beta):
    del padded_neighbor_list
    b1r = b1.reshape(1, D)
    b2r = b2.reshape(1, D)
    g1 = bn1_gamma.reshape(1, D)
    be1 = bn1_beta.reshape(1, D)
    g2 = bn2_gamma.reshape(1, D)
    be2 = bn2_beta.reshape(1, D)
    const = lambda t: (0, 0)

    out = pl.pallas_call(
        _gin_kernel,
        grid=(S1 + 2 * S2,),
        in_specs=[
            pl.BlockSpec((N, D), const),
            pl.BlockSpec((TM, N), lambda t: (jnp.minimum(t, S1 - 1), 0)),
            pl.BlockSpec((D, D), const),
            pl.BlockSpec((1, D), const),
            pl.BlockSpec((1, D), const),
            pl.BlockSpec((1, D), const),
            pl.BlockSpec((D, D), const),
            pl.BlockSpec((1, D), const),
            pl.BlockSpec((1, D), const),
            pl.BlockSpec((1, D), const),
        ],
        out_specs=pl.BlockSpec(
            (TM2, D), lambda t: (jnp.maximum(t - (S1 + S2), 0), 0)),
        out_shape=jax.ShapeDtypeStruct((N, D), jnp.float32),
        scratch_shapes=[
            pltpu.VMEM((N, D), jnp.float32),
            pltpu.VMEM((N, D), jnp.float32),
            pltpu.VMEM((4, D), jnp.float32),
        ],
    )(h, Adj_block, W1, b1r, g1, be1, W2, b2r, g2, be2)

    return out
